# baseline (speedup 1.0000x reference)
import functools

import jax
import jax.numpy as jnp
from jax import lax
from jax.experimental import pallas as pl
from jax.experimental.pallas import tpu as pltpu
from jax.experimental.pallas import tpu_sc as plsc

VOCAB_N = 1000000
EMBED_N = 64
CTX_N = 16384

SC_CORES = 2
SC_SUBCORES = 16
SC_WORKERS = SC_CORES * SC_SUBCORES
IDX_PER_W = CTX_N // SC_WORKERS
GATHER_CHUNK = 128

BLK = 65536
NBLK = (VOCAB_N + BLK - 1) // BLK
BLKC = 65536
NBLKC = (VOCAB_N + BLKC - 1) // BLKC


_STRIPE = 62512
_STRIPE_LAST = VOCAB_N - 15 * _STRIPE
_IDX_CHUNK = 128
_N_CHUNK = IDX_PER_W // _IDX_CHUNK


@functools.lru_cache(maxsize=None)
def _build_sc_counts():
    mesh = plsc.VectorSubcoreMesh(core_axis_name="c", subcore_axis_name="s")

    @functools.partial(
        pl.kernel,
        mesh=mesh,
        out_type=jax.ShapeDtypeStruct((SC_CORES, VOCAB_N), jnp.float32),
        scratch_types=[
            pltpu.VMEM((_N_CHUNK, _IDX_CHUNK), jnp.int32),
            pltpu.VMEM((_IDX_CHUNK,), jnp.float32),
            pltpu.VMEM((_STRIPE,), jnp.float32),
            pltpu.VMEM_SHARED((VOCAB_N,), jnp.float32),
        ],
    )
    def sc_kernel(x_hbm, out_hbm, idx_v, ones_v, zeros_v, shared):
        cid = lax.axis_index("c")
        sid = lax.axis_index("s")
        base = cid * (CTX_N // 2) + sid * IDX_PER_W
        for c4 in range(_N_CHUNK):
            pltpu.sync_copy(
                x_hbm.at[pl.ds(base + c4 * _IDX_CHUNK, _IDX_CHUNK)],
                idx_v.at[c4],
            )
        zvec = jnp.zeros((16,), jnp.float32)
        onev = jnp.ones((16,), jnp.float32)
        for j in range(_IDX_CHUNK // 16):
            ones_v[pl.ds(j * 16, 16)] = onev

        def zbody(i, _):
            zeros_v[pl.ds(i * 16, 16)] = zvec
            return 0

        lax.fori_loop(0, _STRIPE // 16, zbody, 0)

        @pl.when(sid < 15)
        def _():
            pltpu.sync_copy(zeros_v, shared.at[pl.ds(sid * _STRIPE, _STRIPE)])

        @pl.when(sid == 15)
        def _():
            pltpu.sync_copy(
                zeros_v.at[pl.ds(0, _STRIPE_LAST)],
                shared.at[pl.ds(15 * _STRIPE, _STRIPE_LAST)],
            )

        plsc.subcore_barrier()
        for c4 in range(_N_CHUNK):
            pltpu.sync_copy(ones_v, shared.at[idx_v.at[c4]], add=True)
        plsc.subcore_barrier()

        @pl.when(sid == 0)
        def _():
            pltpu.sync_copy(shared, out_hbm.at[cid])

    return sc_kernel


def _mean_kernel(counts_ref, embt_ref, out_ref):
    i = pl.program_id(0)
    cols = lax.broadcasted_iota(jnp.int32, (1, BLKC), 1) + i * BLKC
    valid = cols < VOCAB_N
    cb = jnp.where(
        valid, counts_ref[0:1, :] + counts_ref[1:2, :], jnp.float32(0.0)
    )
    embm = jnp.where(valid, embt_ref[...], jnp.float32(0.0))
    part = lax.dot_general(
        cb, embm, (((1,), (1,)), ((), ())),
        preferred_element_type=jnp.float32,
    )

    @pl.when(i == 0)
    def _():
        out_ref[...] = jnp.zeros_like(out_ref)

    acc = out_ref[...] + part

    @pl.when(i == NBLKC - 1)
    def _():
        out_ref[...] = acc * jnp.float32(1.0 / CTX_N)

    @pl.when(i < NBLKC - 1)
    def _():
        out_ref[...] = acc


def _logits_kernel(mean_ref, w_ref, b_ref, out_ref, c_ref, m_ref, s_ref):
    i = pl.program_id(0)
    mean = mean_ref[...]
    logits = lax.dot_general(
        mean, w_ref[...], (((1,), (0,)), ((), ())),
        preferred_element_type=jnp.float32,
    ) + b_ref[...]
    cols = lax.broadcasted_iota(jnp.int32, (1, BLK), 1) + i * BLK
    valid = cols < VOCAB_N
    neg_inf = jnp.float32(-jnp.inf)
    lm = jnp.where(valid, logits, neg_inf)
    out_ref[...] = logits
    bm = jnp.max(lm)
    m_prev = jnp.where(i == 0, neg_inf, m_ref[0])
    s_prev = jnp.where(i == 0, jnp.float32(0.0), s_ref[0])
    m_new = jnp.maximum(m_prev, bm)
    s_new = s_prev * jnp.exp(m_prev - m_new) + jnp.sum(
        jnp.where(valid, jnp.exp(lm - m_new), jnp.float32(0.0))
    )
    m_ref[0] = m_new
    s_ref[0] = s_new

    @pl.when(i == NBLK - 1)
    def _():
        c_ref[0, 0] = m_new + jnp.log(s_new)


def _normalize_kernel(logits_ref, c_ref, out_ref):
    out_ref[...] = logits_ref[...] - c_ref[0, 0]


@functools.lru_cache(maxsize=None)
def _build_tc_calls(interpret: bool = False):
    mean_call = pl.pallas_call(
        _mean_kernel,
        grid=(NBLKC,),
        in_specs=[
            pl.BlockSpec((SC_CORES, BLKC), lambda i: (0, i)),
            pl.BlockSpec((EMBED_N, BLKC), lambda i: (0, i)),
        ],
        out_specs=pl.BlockSpec((1, EMBED_N), lambda i: (0, 0)),
        out_shape=jax.ShapeDtypeStruct((1, EMBED_N), jnp.float32),
        interpret=interpret,
    )
    logits_call = pl.pallas_call(
        _logits_kernel,
        grid=(NBLK,),
        in_specs=[
            pl.BlockSpec((1, EMBED_N), lambda i: (0, 0)),
            pl.BlockSpec((EMBED_N, BLK), lambda i: (0, i)),
            pl.BlockSpec((1, BLK), lambda i: (0, i)),
        ],
        out_specs=[
            pl.BlockSpec((1, BLK), lambda i: (0, i)),
            pl.BlockSpec((1, 1), lambda i: (0, 0), memory_space=pltpu.SMEM),
        ],
        out_shape=[
            jax.ShapeDtypeStruct((1, VOCAB_N), jnp.float32),
            jax.ShapeDtypeStruct((1, 1), jnp.float32),
        ],
        scratch_shapes=[
            pltpu.SMEM((1,), jnp.float32),
            pltpu.SMEM((1,), jnp.float32),
        ],
        interpret=interpret,
    )
    norm_call = pl.pallas_call(
        _normalize_kernel,
        grid=(NBLK,),
        in_specs=[
            pl.BlockSpec((1, BLK), lambda i: (0, i)),
            pl.BlockSpec((1, 1), lambda i: (0, 0), memory_space=pltpu.SMEM),
        ],
        out_specs=pl.BlockSpec((1, BLK), lambda i: (0, i)),
        out_shape=jax.ShapeDtypeStruct((1, VOCAB_N), jnp.float32),
        interpret=interpret,
    )
    return mean_call, logits_call, norm_call


def kernel(X, embedding, W, b):
    counts = _build_sc_counts()(X)
    mean_call, logits_call, norm_call = _build_tc_calls()
    mean = mean_call(counts, embedding.T)
    logits, c = logits_call(mean, W.T, b.reshape(1, VOCAB_N))
    return norm_call(logits, c)
